# Initial kernel scaffold; baseline (speedup 1.0000x reference)
import jax
import jax.numpy as jnp
from jax.experimental import pallas as pl
from jax.experimental.pallas import tpu as pltpu

_K = 8
_BLK = 512


def _router_block(x_ref, wt_ref, fw_ref, idx_ref):
    blk, e = fw_ref.shape
    logits = jax.lax.dot_general(
        x_ref[...], wt_ref[...], (((1,), (0,)), ((), ())),
        preferred_element_type=jnp.float32,
        precision=jax.lax.Precision.HIGHEST)
    iota = jax.lax.broadcasted_iota(jnp.int32, (blk, e), 1)
    cur = logits
    sel = jnp.zeros((blk, e), jnp.bool_)
    idx_cols = []
    m0 = None
    for k in range(_K):
        m = jnp.max(cur, axis=1, keepdims=True)
        if k == 0:
            m0 = m
        amax = jnp.min(jnp.where(cur == m, iota, e), axis=1, keepdims=True)
        idx_cols.append(amax)
        hit = iota == amax
        sel = jnp.logical_or(sel, hit)
        cur = jnp.where(hit, -jnp.inf, cur)
    ex = jnp.where(sel, jnp.exp(logits - m0), 0.0)
    z = jnp.sum(ex, axis=1, keepdims=True)
    fw_ref[...] = ex / z
    idx_ref[...] = jnp.concatenate(idx_cols, axis=1)


def kernel(x, W):
    b, d = x.shape
    e = W.shape[0]
    wt = W.T
    fw, idx = pl.pallas_call(
        _router_block,
        grid=(b // _BLK,),
        in_specs=[
            pl.BlockSpec((_BLK, d), lambda i: (i, 0)),
            pl.BlockSpec((d, e), lambda i: (0, 0)),
        ],
        out_specs=[
            pl.BlockSpec((_BLK, e), lambda i: (i, 0)),
            pl.BlockSpec((_BLK, _K), lambda i: (i, 0)),
        ],
        out_shape=[
            jax.ShapeDtypeStruct((b, e), jnp.float32),
            jax.ShapeDtypeStruct((b, _K), jnp.int32),
        ],
        compiler_params=pltpu.CompilerParams(
            dimension_semantics=("arbitrary",)),
    )(x, wt)
    return fw, idx


# speedup vs baseline: 4.9941x; 4.9941x over previous
import jax
import jax.numpy as jnp
from jax.experimental import pallas as pl
from jax.experimental.pallas import tpu as pltpu

_K = 8
_BLK = 512


def _router_block(x_ref, wt_ref, fw_ref, idx_ref):
    blk, e = fw_ref.shape
    logits = jax.lax.dot_general(
        x_ref[...].astype(jnp.bfloat16), wt_ref[...].astype(jnp.bfloat16),
        (((1,), (0,)), ((), ())),
        preferred_element_type=jnp.float32,
        precision=jax.lax.Precision.DEFAULT)
    iota = jax.lax.broadcasted_iota(jnp.int32, (blk, e), 1)
    cur = logits
    sel = jnp.zeros((blk, e), jnp.bool_)
    idx_cols = []
    m0 = None
    for k in range(_K):
        m = jnp.max(cur, axis=1, keepdims=True)
        if k == 0:
            m0 = m
        amax = jnp.min(jnp.where(cur == m, iota, e), axis=1, keepdims=True)
        idx_cols.append(amax)
        hit = iota == amax
        sel = jnp.logical_or(sel, hit)
        cur = jnp.where(hit, -jnp.inf, cur)
    ex = jnp.where(sel, jnp.exp(logits - m0), 0.0)
    z = jnp.sum(ex, axis=1, keepdims=True)
    fw_ref[...] = ex / z
    idx_ref[...] = jnp.concatenate(idx_cols, axis=1)


def kernel(x, W):
    b, d = x.shape
    e = W.shape[0]
    wt = W.T
    fw, idx = pl.pallas_call(
        _router_block,
        grid=(b // _BLK,),
        in_specs=[
            pl.BlockSpec((_BLK, d), lambda i: (i, 0)),
            pl.BlockSpec((d, e), lambda i: (0, 0)),
        ],
        out_specs=[
            pl.BlockSpec((_BLK, e), lambda i: (i, 0)),
            pl.BlockSpec((_BLK, _K), lambda i: (i, 0)),
        ],
        out_shape=[
            jax.ShapeDtypeStruct((b, e), jnp.float32),
            jax.ShapeDtypeStruct((b, _K), jnp.int32),
        ],
        compiler_params=pltpu.CompilerParams(
            dimension_semantics=("arbitrary",)),
    )(x, wt)
    return fw, idx


# speedup vs baseline: 5.5520x; 1.1117x over previous
import jax
import jax.numpy as jnp
from jax.experimental import pallas as pl
from jax.experimental.pallas import tpu as pltpu

_K = 8
_BLK = 512


def _router_block(x_ref, wt_ref, fw_ref, idx_ref):
    blk, e = fw_ref.shape
    logits = jax.lax.dot_general(
        x_ref[...].astype(jnp.bfloat16), wt_ref[...].astype(jnp.bfloat16),
        (((1,), (0,)), ((), ())),
        preferred_element_type=jnp.float32,
        precision=jax.lax.Precision.DEFAULT)
    iota = jax.lax.broadcasted_iota(
        jnp.int32, (blk, e), 1).astype(jnp.float32)
    cur = logits
    sel = jnp.zeros((blk, e), jnp.bool_)
    idx_cols = []
    m0 = None
    for k in range(_K):
        m = jnp.max(cur, axis=1, keepdims=True)
        if k == 0:
            m0 = m
        amax = jnp.min(jnp.where(cur == m, iota, float(e)), axis=1,
                       keepdims=True)
        idx_cols.append(amax)
        hit = iota == amax
        sel = jnp.logical_or(sel, hit)
        cur = jnp.where(hit, -jnp.inf, cur)
    ex = jnp.where(sel, jnp.exp(logits - m0), 0.0)
    z = jnp.sum(ex, axis=1, keepdims=True)
    fw_ref[...] = ex / z
    idx_ref[...] = jnp.concatenate(idx_cols, axis=1).astype(jnp.int32)


def kernel(x, W):
    b, d = x.shape
    e = W.shape[0]
    wt = W.T
    fw, idx = pl.pallas_call(
        _router_block,
        grid=(b // _BLK,),
        in_specs=[
            pl.BlockSpec((_BLK, d), lambda i: (i, 0)),
            pl.BlockSpec((d, e), lambda i: (0, 0)),
        ],
        out_specs=[
            pl.BlockSpec((_BLK, e), lambda i: (i, 0)),
            pl.BlockSpec((_BLK, _K), lambda i: (i, 0)),
        ],
        out_shape=[
            jax.ShapeDtypeStruct((b, e), jnp.float32),
            jax.ShapeDtypeStruct((b, _K), jnp.int32),
        ],
        compiler_params=pltpu.CompilerParams(
            dimension_semantics=("arbitrary",)),
    )(x, wt)
    return fw, idx


# speedup vs baseline: 5.5522x; 1.0000x over previous
import jax
import jax.numpy as jnp
from jax.experimental import pallas as pl
from jax.experimental.pallas import tpu as pltpu

_K = 8
_BLK = 512


def _router_block(x_ref, wt_ref, fw_ref, idx_ref):
    blk, e = fw_ref.shape
    logits = jax.lax.dot_general(
        x_ref[...].astype(jnp.bfloat16), wt_ref[...].astype(jnp.bfloat16),
        (((1,), (0,)), ((), ())),
        preferred_element_type=jnp.float32,
        precision=jax.lax.Precision.DEFAULT)
    iota = jax.lax.broadcasted_iota(
        jnp.int32, (blk, e), 1).astype(jnp.float32)
    cur = logits
    sel = jnp.zeros((blk, e), jnp.bool_)
    idx_cols = []
    m0 = None
    for k in range(_K):
        m = jnp.max(cur, axis=1, keepdims=True)
        if k == 0:
            m0 = m
        amax = jnp.min(jnp.where(cur == m, iota, float(e)), axis=1,
                       keepdims=True)
        idx_cols.append(amax)
        hit = iota == amax
        sel = jnp.logical_or(sel, hit)
        cur = jnp.where(hit, -jnp.inf, cur)
    ex = jnp.where(sel, jnp.exp(logits - m0), 0.0)
    z = jnp.sum(ex, axis=1, keepdims=True)
    fw_ref[...] = ex / z
    idx_ref[...] = jnp.concatenate(idx_cols, axis=1).astype(jnp.int32)


def kernel(x, W):
    b, d = x.shape
    e = W.shape[0]
    wt = W.T
    fw, idx = pl.pallas_call(
        _router_block,
        grid=(b // _BLK,),
        in_specs=[
            pl.BlockSpec((_BLK, d), lambda i: (i, 0)),
            pl.BlockSpec((d, e), lambda i: (0, 0)),
        ],
        out_specs=[
            pl.BlockSpec((_BLK, e), lambda i: (i, 0)),
            pl.BlockSpec((_BLK, _K), lambda i: (i, 0)),
        ],
        out_shape=[
            jax.ShapeDtypeStruct((b, e), jnp.float32),
            jax.ShapeDtypeStruct((b, _K), jnp.int32),
        ],
        compiler_params=pltpu.CompilerParams(
            dimension_semantics=("parallel",)),
    )(x, wt)
    return fw, idx


# speedup vs baseline: 5.6904x; 1.0249x over previous
import jax
import jax.numpy as jnp
from jax.experimental import pallas as pl
from jax.experimental.pallas import tpu as pltpu

_K = 8
_BLK = 512


def _router_block(x_ref, w_ref, fw_ref, idx_ref):
    blk, e = fw_ref.shape
    logits = jax.lax.dot_general(
        x_ref[...].astype(jnp.bfloat16), w_ref[...].astype(jnp.bfloat16),
        (((1,), (1,)), ((), ())),
        preferred_element_type=jnp.float32,
        precision=jax.lax.Precision.DEFAULT)
    iota = jax.lax.broadcasted_iota(
        jnp.int32, (blk, e), 1).astype(jnp.float32)
    cur = logits
    idx_cols = []
    m0 = None
    for k in range(_K):
        m = jnp.max(cur, axis=1, keepdims=True)
        if k == 0:
            m0 = m
        amax = jnp.min(jnp.where(cur == m, iota, float(e)), axis=1,
                       keepdims=True)
        idx_cols.append(amax)
        cur = jnp.where(iota == amax, -jnp.inf, cur)
    sel = cur == -jnp.inf
    ex = jnp.where(sel, jnp.exp(logits - m0), 0.0)
    z = jnp.sum(ex, axis=1, keepdims=True)
    fw_ref[...] = ex / z
    idx_ref[...] = jnp.concatenate(idx_cols, axis=1).astype(jnp.int32)


def kernel(x, W):
    b, d = x.shape
    e = W.shape[0]
    fw, idx = pl.pallas_call(
        _router_block,
        grid=(b // _BLK,),
        in_specs=[
            pl.BlockSpec((_BLK, d), lambda i: (i, 0)),
            pl.BlockSpec((e, d), lambda i: (0, 0)),
        ],
        out_specs=[
            pl.BlockSpec((_BLK, e), lambda i: (i, 0)),
            pl.BlockSpec((_BLK, _K), lambda i: (i, 0)),
        ],
        out_shape=[
            jax.ShapeDtypeStruct((b, e), jnp.float32),
            jax.ShapeDtypeStruct((b, _K), jnp.int32),
        ],
        compiler_params=pltpu.CompilerParams(
            dimension_semantics=("parallel",)),
    )(x, W)
    return fw, idx


# speedup vs baseline: 6.3558x; 1.1169x over previous
import jax
import jax.numpy as jnp
from jax.experimental import pallas as pl
from jax.experimental.pallas import tpu as pltpu

_K = 8
_BLK = 1024


def _router_block(x_ref, w_ref, fw_ref, idx_ref):
    blk, e = fw_ref.shape
    logits = jax.lax.dot_general(
        x_ref[...].astype(jnp.bfloat16), w_ref[...].astype(jnp.bfloat16),
        (((1,), (1,)), ((), ())),
        preferred_element_type=jnp.float32,
        precision=jax.lax.Precision.DEFAULT)
    iota = jax.lax.broadcasted_iota(
        jnp.int32, (blk, e), 1).astype(jnp.float32)
    cur = logits
    idx_cols = []
    m0 = None
    for k in range(_K):
        m = jnp.max(cur, axis=1, keepdims=True)
        if k == 0:
            m0 = m
        amax = jnp.min(jnp.where(cur == m, iota, float(e)), axis=1,
                       keepdims=True)
        idx_cols.append(amax)
        cur = jnp.where(iota == amax, -jnp.inf, cur)
    sel = cur == -jnp.inf
    ex = jnp.where(sel, jnp.exp(logits - m0), 0.0)
    z = jnp.sum(ex, axis=1, keepdims=True)
    fw_ref[...] = ex / z
    idx_ref[...] = jnp.concatenate(idx_cols, axis=1).astype(jnp.int32)


def kernel(x, W):
    b, d = x.shape
    e = W.shape[0]
    fw, idx = pl.pallas_call(
        _router_block,
        grid=(b // _BLK,),
        in_specs=[
            pl.BlockSpec((_BLK, d), lambda i: (i, 0)),
            pl.BlockSpec((e, d), lambda i: (0, 0)),
        ],
        out_specs=[
            pl.BlockSpec((_BLK, e), lambda i: (i, 0)),
            pl.BlockSpec((_BLK, _K), lambda i: (i, 0)),
        ],
        out_shape=[
            jax.ShapeDtypeStruct((b, e), jnp.float32),
            jax.ShapeDtypeStruct((b, _K), jnp.int32),
        ],
        compiler_params=pltpu.CompilerParams(
            dimension_semantics=("parallel",)),
    )(x, W)
    return fw, idx
